# baseline (speedup 1.0000x reference)
import jax
import jax.numpy as jnp
from jax import lax
from jax.experimental import pallas as pl
from jax.experimental.pallas import tpu as pltpu
from jax.experimental.pallas import tpu_sc as plsc

CAP = 1000000
FEAT = 32
B = 16384
NC = 2
NS = 16
L = 16
NW = NC * NS
NA = 24
RPA = 31248
RPB = 31256
CW = 30720
NB = 4
NFULL = 32
TA = RPA * FEAT - NFULL * CW
TB = RPB * FEAT - NFULL * CW
NIDX = B // L
CAPL = 1024
LISTN = CAPL + 2 * L
PSHIFT = 16384


def _body(bxf, by, xf, y, idx, obxf, oby,
          llist, plist, wl, wp, pk, mbuf,
          sem_in0, sem_in1, sem_in2, sem_in3,
          sem_out0, sem_out1, sem_out2, sem_out3, sem_s):
  wid = lax.axis_index("s") * NC + lax.axis_index("c")
  base = wid * RPA + jnp.maximum(wid - NA, 0) * (RPB - RPA)
  is_b = wid >= NA
  rpw = jnp.where(is_b, RPB, RPA)
  iota = lax.iota(jnp.int32, L)
  zeros = jnp.zeros((L,), jnp.int32)

  def _phase1(u_buf, tab, by_buf):
    pltpu.sync_copy(idx, u_buf)

    def _zero(j, _):
      llist[pl.ds(j * L, L)] = zeros
      plist[pl.ds(j * L, L)] = zeros
      return 0
    lax.fori_loop(0, LISTN // L, _zero, 0)

    def _filter(k, cnt):
      v = u_buf[pl.ds(k * L, L)]
      inr = jnp.logical_and(v >= base, v < base + rpw)
      pos = k * L + iota
      inr_i = inr.astype(jnp.int32)
      cum = plsc.cumsum(inr_i)
      offs = cnt + cum - inr_i
      plsc.store_scatter(llist, [offs], v - base, mask=inr)
      plsc.store_scatter(plist, [offs], pos, mask=inr)
      return jnp.minimum(cnt + cum[L - 1], CAPL)
    n = lax.fori_loop(0, NIDX, _filter, jnp.int32(0))

    def _ded1(g, _):
      lanes = g * L + iota
      valid = lanes < n
      iv = llist[pl.ds(g * L, L)]
      pv = plist[pl.ds(g * L, L)]
      for l in range(L):
        plsc.store_scatter(tab, [iv], pv,
                           mask=jnp.logical_and(valid, iota == l))
      return 0
    lax.fori_loop(0, (n + L - 1) // L, _ded1, 0)

    def _ded2(g, m):
      lanes = g * L + iota
      valid = lanes < n
      iv = llist[pl.ds(g * L, L)]
      pv = plist[pl.ds(g * L, L)]
      w = plsc.load_gather(tab, [iv], mask=valid)
      win = jnp.logical_and(valid, w == pv)
      win_i = win.astype(jnp.int32)
      cum = plsc.cumsum(win_i)
      offs = m + cum - win_i
      plsc.store_scatter(wl, [offs], iv, mask=win)
      plsc.store_scatter(wp, [offs], pv, mask=win)
      return jnp.minimum(m + cum[L - 1], CAPL)
    m = lax.fori_loop(0, (n + L - 1) // L, _ded2, jnp.int32(0))

    mbuf[pl.ds(0, L)] = jnp.where(iota == 0, m, 0)

    @pl.when(jnp.logical_not(is_b))
    def _():
      pltpu.sync_copy(by.at[pl.ds(base, RPA)], by_buf.at[pl.ds(0, RPA)])

    @pl.when(is_b)
    def _():
      pltpu.sync_copy(by.at[pl.ds(base, RPB)], by_buf.at[pl.ds(0, RPB)])

    @pl.when(m > 0)
    def _():
      def _pack(g, _):
        lv = wl[pl.ds(g * L, L)]
        pv = wp[pl.ds(g * L, L)]
        pk[pl.ds(g * L, L)] = jnp.bitwise_or(lv * PSHIFT, pv)
        return 0
      lax.fori_loop(0, (m + L - 1) // L, _pack, 0)

      pltpu.sync_copy(y, u_buf)

      def _appy(g, _):
        lanes = g * L + iota
        msk = lanes < m
        iv = wl[pl.ds(g * L, L)]
        pv = wp[pl.ds(g * L, L)]
        yvv = plsc.load_gather(u_buf, [pv], mask=msk)
        plsc.store_scatter(by_buf, [iv], yvv, mask=msk)
        return 0
      lax.fori_loop(0, (m + L - 1) // L, _appy, 0)

    @pl.when(jnp.logical_not(is_b))
    def _():
      pltpu.sync_copy(by_buf.at[pl.ds(0, RPA)], oby.at[pl.ds(base, RPA)])

    @pl.when(is_b)
    def _():
      pltpu.sync_copy(by_buf.at[pl.ds(0, RPB)], oby.at[pl.ds(base, RPB)])

  pl.run_scoped(_phase1,
                pltpu.VMEM((B,), jnp.int32),
                pltpu.VMEM((RPB,), jnp.int32),
                pltpu.VMEM((RPB,), jnp.int32))

  m = mbuf[pl.ds(0, L)][0]

  fbase = base * FEAT

  def _phase2(*bufs_and_sems):
    bufs = bufs_and_sems[:NB]
    isems = [sem_in0, sem_in1, sem_in2, sem_in3]
    osems = [sem_out0, sem_out1, sem_out2, sem_out3]

    def _wait_out(bi):
      pltpu.make_async_copy(bufs[bi], obxf.at[pl.ds(fbase, CW)],
                            osems[bi]).wait()

    def _round(r, _):
      o = fbase + (NB * r) * CW
      for bi in range(NB):
        @pl.when(r > 0)
        def _():
          _wait_out(bi)
        pltpu.async_copy(bxf.at[pl.ds(o + bi * CW, CW)], bufs[bi], isems[bi])
      for bi in range(NB):
        pltpu.make_async_copy(bxf.at[pl.ds(o + bi * CW, CW)],
                              bufs[bi], isems[bi]).wait()
        pltpu.async_copy(bufs[bi], obxf.at[pl.ds(o + bi * CW, CW)], osems[bi])
      return 0
    lax.fori_loop(0, NFULL // NB, _round, 0)

    for bi in range(NB):
      _wait_out(bi)

    ot = fbase + NFULL * CW

    @pl.when(jnp.logical_not(is_b))
    def _():
      tsrc = bxf.at[pl.ds(ot, TA)]
      tdst = obxf.at[pl.ds(ot, TA)]
      tbuf = bufs[0].at[pl.ds(0, TA)]
      pltpu.async_copy(tsrc, tbuf, sem_in0)
      pltpu.make_async_copy(tsrc, tbuf, sem_in0).wait()
      pltpu.async_copy(tbuf, tdst, sem_out0)
      pltpu.make_async_copy(tbuf, tdst, sem_out0).wait()

    @pl.when(is_b)
    def _():
      tsrc = bxf.at[pl.ds(ot, TB)]
      tdst = obxf.at[pl.ds(ot, TB)]
      tbuf = bufs[0].at[pl.ds(0, TB)]
      pltpu.async_copy(tsrc, tbuf, sem_in0)
      pltpu.make_async_copy(tsrc, tbuf, sem_in0).wait()
      pltpu.async_copy(tbuf, tdst, sem_out0)
      pltpu.make_async_copy(tbuf, tdst, sem_out0).wait()

  pl.run_scoped(_phase2,
                pltpu.VMEM((CW,), jnp.float32),
                pltpu.VMEM((CW,), jnp.float32),
                pltpu.VMEM((CW,), jnp.float32),
                pltpu.VMEM((CW,), jnp.float32))

  @pl.when(m > 0)
  def _():
    def _phase3(rowbuf):
      def _blk(g, _):
        vec = pk[pl.ds(g * L, L)]
        for k in range(L):
          @pl.when(g * L + k < m)
          def _():
            e = vec[k]
            p = jax.lax.rem(e, PSHIFT)
            slot = (g * L + k) * FEAT
            pltpu.async_copy(xf.at[pl.ds(p * FEAT, FEAT)],
                             rowbuf.at[pl.ds(slot, FEAT)], sem_s)
        for k in range(L):
          @pl.when(g * L + k < m)
          def _():
            e = vec[k]
            p = jax.lax.rem(e, PSHIFT)
            slot = (g * L + k) * FEAT
            pltpu.make_async_copy(xf.at[pl.ds(p * FEAT, FEAT)],
                                  rowbuf.at[pl.ds(slot, FEAT)], sem_s).wait()
        for k in range(L):
          @pl.when(g * L + k < m)
          def _():
            e = vec[k]
            r = jax.lax.div(e, PSHIFT)
            slot = (g * L + k) * FEAT
            pltpu.async_copy(rowbuf.at[pl.ds(slot, FEAT)],
                             obxf.at[pl.ds((base + r) * FEAT, FEAT)], sem_s)
        for k in range(L):
          @pl.when(g * L + k < m)
          def _():
            e = vec[k]
            r = jax.lax.div(e, PSHIFT)
            slot = (g * L + k) * FEAT
            pltpu.make_async_copy(rowbuf.at[pl.ds(slot, FEAT)],
                                  obxf.at[pl.ds((base + r) * FEAT, FEAT)],
                                  sem_s).wait()
        return 0
      lax.fori_loop(0, (m + L - 1) // L, _blk, 0)

    pl.run_scoped(_phase3, pltpu.VMEM((CAPL * FEAT,), jnp.float32))


_mesh = plsc.VectorSubcoreMesh(core_axis_name="c", subcore_axis_name="s",
                               num_cores=NC, num_subcores=NS)

_sc_call = pl.kernel(
    _body,
    out_type=(jax.ShapeDtypeStruct((CAP * FEAT,), jnp.float32),
              jax.ShapeDtypeStruct((CAP,), jnp.int32)),
    mesh=_mesh,
    compiler_params=pltpu.CompilerParams(needs_layout_passes=False),
    scratch_types=[
        pltpu.VMEM((LISTN,), jnp.int32),
        pltpu.VMEM((LISTN,), jnp.int32),
        pltpu.VMEM((LISTN,), jnp.int32),
        pltpu.VMEM((LISTN,), jnp.int32),
        pltpu.VMEM((LISTN,), jnp.int32),
        pltpu.VMEM((L,), jnp.int32),
    ] + [pltpu.SemaphoreType.DMA] * 9,
)


def kernel(bx, by, x, y, idx):
  obxf, oby = _sc_call(bx.reshape(CAP * FEAT), by, x.reshape(B * FEAT), y, idx)
  return obxf.reshape(CAP, FEAT), oby


# speedup vs baseline: 1.0205x; 1.0205x over previous
import jax
import jax.numpy as jnp
from jax import lax
from jax.experimental import pallas as pl
from jax.experimental.pallas import tpu as pltpu
from jax.experimental.pallas import tpu_sc as plsc

CAP = 1000000
FEAT = 32
B = 16384
NC = 2
NS = 16
L = 16
NW = NC * NS
NA = 24
RPA = 31248
RPB = 31256
NIDX = B // L
CAPL = 1024
LISTN = CAPL + 2 * L
PSHIFT = 16384
ROWSLOTS = 512

CW = 224000
NB = 4
NFULL = 71
SC0W = 16 * RPA * FEAT
SC1W = CAP * FEAT - SC0W
TSC0 = SC0W - NFULL * CW
TSC1 = SC1W - NFULL * CW


def _body(bxf, by, xf, y, idx, obxf, oby,
          llist, plist, wl, wp, pk, ywin, u_buf, big_buf, rowbuf, shbuf,
          sem_in0, sem_in1, sem_in2, sem_in3,
          sem_out0, sem_out1, sem_out2, sem_out3, sem_s):
  sid = lax.axis_index("s")
  cid = lax.axis_index("c")
  wid = cid * NS + sid
  base = wid * RPA + jnp.maximum(wid - NA, 0) * (RPB - RPA)
  is_b = wid >= NA
  rpw = jnp.where(is_b, RPB, RPA)
  iota = lax.iota(jnp.int32, L)
  zeros = jnp.zeros((L,), jnp.int32)

  tab = big_buf
  pltpu.sync_copy(idx, u_buf)

  def _zero(j, _):
    llist[pl.ds(j * L, L)] = zeros
    plist[pl.ds(j * L, L)] = zeros
    return 0
  lax.fori_loop(0, LISTN // L, _zero, 0)

  def _filter(k, cnt):
    v = u_buf[pl.ds(k * L, L)]
    inr = jnp.logical_and(v >= base, v < base + rpw)
    pos = k * L + iota
    inr_i = inr.astype(jnp.int32)
    cum = plsc.cumsum(inr_i)
    offs = cnt + cum - inr_i
    plsc.store_scatter(llist, [offs], v - base, mask=inr)
    plsc.store_scatter(plist, [offs], pos, mask=inr)
    return jnp.minimum(cnt + cum[L - 1], CAPL)
  n = lax.fori_loop(0, NIDX, _filter, jnp.int32(0))

  def _ded1(g, _):
    lanes = g * L + iota
    valid = lanes < n
    iv = llist[pl.ds(g * L, L)]
    pv = plist[pl.ds(g * L, L)]
    for l in range(L):
      plsc.store_scatter(tab, [iv], pv,
                         mask=jnp.logical_and(valid, iota == l))
    return 0
  lax.fori_loop(0, (n + L - 1) // L, _ded1, 0)

  def _ded2(g, m):
    lanes = g * L + iota
    valid = lanes < n
    iv = llist[pl.ds(g * L, L)]
    pv = plist[pl.ds(g * L, L)]
    w = plsc.load_gather(tab, [iv], mask=valid)
    win = jnp.logical_and(valid, w == pv)
    win_i = win.astype(jnp.int32)
    cum = plsc.cumsum(win_i)
    offs = m + cum - win_i
    plsc.store_scatter(wl, [offs], iv, mask=win)
    plsc.store_scatter(wp, [offs], pv, mask=win)
    return jnp.minimum(m + cum[L - 1], CAPL)
  m = lax.fori_loop(0, (n + L - 1) // L, _ded2, jnp.int32(0))

  @pl.when(m > 0)
  def _():
    def _pack(g, _):
      lv = wl[pl.ds(g * L, L)]
      pv = wp[pl.ds(g * L, L)]
      pk[pl.ds(g * L, L)] = jnp.bitwise_or(lv * PSHIFT, pv)
      return 0
    lax.fori_loop(0, (m + L - 1) // L, _pack, 0)

    pltpu.sync_copy(y, u_buf)

    def _ygat(g, _):
      lanes = g * L + iota
      msk = lanes < m
      pv = wp[pl.ds(g * L, L)]
      ywin[pl.ds(g * L, L)] = plsc.load_gather(u_buf, [pv], mask=msk)
      return 0
    lax.fori_loop(0, (m + L - 1) // L, _ygat, 0)

  by_buf = big_buf

  @pl.when(jnp.logical_not(is_b))
  def _():
    pltpu.sync_copy(by.at[pl.ds(base, RPA)], by_buf.at[pl.ds(0, RPA)])

  @pl.when(is_b)
  def _():
    pltpu.sync_copy(by.at[pl.ds(base, RPB)], by_buf.at[pl.ds(0, RPB)])

  @pl.when(m > 0)
  def _():
    def _appy(g, _):
      lanes = g * L + iota
      msk = lanes < m
      iv = wl[pl.ds(g * L, L)]
      yvv = ywin[pl.ds(g * L, L)]
      plsc.store_scatter(by_buf, [iv], yvv, mask=msk)
      return 0
    lax.fori_loop(0, (m + L - 1) // L, _appy, 0)

  @pl.when(jnp.logical_not(is_b))
  def _():
    pltpu.sync_copy(by_buf.at[pl.ds(0, RPA)], oby.at[pl.ds(base, RPA)])

  @pl.when(is_b)
  def _():
    pltpu.sync_copy(by_buf.at[pl.ds(0, RPB)], oby.at[pl.ds(base, RPB)])

  scbase = cid * SC0W
  isems = [sem_in0, sem_in1, sem_in2, sem_in3]
  osems = [sem_out0, sem_out1, sem_out2, sem_out3]

  @pl.when(sid == 0)
  def _():
    def _wait_out(bi):
      pltpu.make_async_copy(shbuf.at[bi], obxf.at[pl.ds(scbase, CW)],
                            osems[bi]).wait()

    def _round(r, _):
      o = scbase + (NB * r) * CW
      for bi in range(NB):
        @pl.when(jnp.logical_and(NB * r + bi < NFULL, r > 0))
        def _():
          _wait_out(bi)

        @pl.when(NB * r + bi < NFULL)
        def _():
          pltpu.async_copy(bxf.at[pl.ds(o + bi * CW, CW)], shbuf.at[bi],
                           isems[bi])
      for bi in range(NB):
        @pl.when(NB * r + bi < NFULL)
        def _():
          pltpu.make_async_copy(bxf.at[pl.ds(o + bi * CW, CW)],
                                shbuf.at[bi], isems[bi]).wait()
          pltpu.async_copy(shbuf.at[bi], obxf.at[pl.ds(o + bi * CW, CW)],
                           osems[bi])
      return 0
    lax.fori_loop(0, (NFULL + NB - 1) // NB, _round, 0)

    ot = scbase + NFULL * CW
    _wait_out(3)

    @pl.when(cid == 0)
    def _():
      tsrc = bxf.at[pl.ds(ot, TSC0)]
      tdst = obxf.at[pl.ds(ot, TSC0)]
      tbuf = shbuf.at[3].at[pl.ds(0, TSC0)]
      pltpu.async_copy(tsrc, tbuf, sem_in3)
      pltpu.make_async_copy(tsrc, tbuf, sem_in3).wait()
      pltpu.async_copy(tbuf, tdst, sem_out3)
      pltpu.make_async_copy(tbuf, tdst, sem_out3).wait()

    @pl.when(cid == 1)
    def _():
      tsrc = bxf.at[pl.ds(ot, TSC1)]
      tdst = obxf.at[pl.ds(ot, TSC1)]
      tbuf = shbuf.at[3].at[pl.ds(0, TSC1)]
      pltpu.async_copy(tsrc, tbuf, sem_in3)
      pltpu.make_async_copy(tsrc, tbuf, sem_in3).wait()
      pltpu.async_copy(tbuf, tdst, sem_out3)
      pltpu.make_async_copy(tbuf, tdst, sem_out3).wait()

    _wait_out(0)
    _wait_out(1)
    _wait_out(2)

  plsc.subcore_barrier()

  @pl.when(m > 0)
  def _():
    def _blk(g, _):
      vec = pk[pl.ds(g * L, L)]
      sbase = (g % (ROWSLOTS // L)) * L * FEAT
      for k in range(L):
        @pl.when(g * L + k < m)
        def _():
          e = vec[k]
          p = jax.lax.rem(e, PSHIFT)
          slot = sbase + k * FEAT
          pltpu.async_copy(xf.at[pl.ds(p * FEAT, FEAT)],
                           rowbuf.at[pl.ds(slot, FEAT)], sem_s)
      for k in range(L):
        @pl.when(g * L + k < m)
        def _():
          e = vec[k]
          p = jax.lax.rem(e, PSHIFT)
          slot = sbase + k * FEAT
          pltpu.make_async_copy(xf.at[pl.ds(p * FEAT, FEAT)],
                                rowbuf.at[pl.ds(slot, FEAT)], sem_s).wait()
      for k in range(L):
        @pl.when(g * L + k < m)
        def _():
          e = vec[k]
          r = jax.lax.div(e, PSHIFT)
          slot = sbase + k * FEAT
          pltpu.async_copy(rowbuf.at[pl.ds(slot, FEAT)],
                           obxf.at[pl.ds((base + r) * FEAT, FEAT)], sem_s)
      for k in range(L):
        @pl.when(g * L + k < m)
        def _():
          e = vec[k]
          r = jax.lax.div(e, PSHIFT)
          slot = sbase + k * FEAT
          pltpu.make_async_copy(rowbuf.at[pl.ds(slot, FEAT)],
                                obxf.at[pl.ds((base + r) * FEAT, FEAT)],
                                sem_s).wait()
      return 0
    lax.fori_loop(0, (m + L - 1) // L, _blk, 0)


_mesh = plsc.VectorSubcoreMesh(core_axis_name="c", subcore_axis_name="s",
                               num_cores=NC, num_subcores=NS)

_sc_call = pl.kernel(
    _body,
    out_type=(jax.ShapeDtypeStruct((CAP * FEAT,), jnp.float32),
              jax.ShapeDtypeStruct((CAP,), jnp.int32)),
    mesh=_mesh,
    compiler_params=pltpu.CompilerParams(needs_layout_passes=False),
    scratch_types=[
        pltpu.VMEM((LISTN,), jnp.int32),
        pltpu.VMEM((LISTN,), jnp.int32),
        pltpu.VMEM((LISTN,), jnp.int32),
        pltpu.VMEM((LISTN,), jnp.int32),
        pltpu.VMEM((LISTN,), jnp.int32),
        pltpu.VMEM((LISTN,), jnp.int32),
        pltpu.VMEM((B,), jnp.int32),
        pltpu.VMEM((RPB,), jnp.int32),
        pltpu.VMEM((ROWSLOTS * FEAT,), jnp.float32),
        pltpu.VMEM_SHARED((NB, CW), jnp.float32),
    ] + [pltpu.SemaphoreType.DMA] * 9,
)


def kernel(bx, by, x, y, idx):
  obxf, oby = _sc_call(bx.reshape(CAP * FEAT), by, x.reshape(B * FEAT), y, idx)
  return obxf.reshape(CAP, FEAT), oby


# speedup vs baseline: 1.5094x; 1.4790x over previous
import jax
import jax.numpy as jnp
from jax import lax
from jax.experimental import pallas as pl
from jax.experimental.pallas import tpu as pltpu
from jax.experimental.pallas import tpu_sc as plsc

CAP = 1000000
FEAT = 32
B = 16384
NC = 2
NS = 16
L = 16
NW = NC * NS
NA = 24
RPA = 31248
RPB = 31256
NIDX = B // L
CAPL = 1024
CAPW = 512
LISTN = CAPL + 2 * L
PSHIFT = 16384


def _sc_body(by, y, idx, oby, pkw, mcnt,
             llist, plist, wl, wp, pk, ywin, u_buf, big_buf, mstage, sem_d):
  sid = lax.axis_index("s")
  cid = lax.axis_index("c")
  wid = cid * NS + sid
  base = wid * RPA + jnp.maximum(wid - NA, 0) * (RPB - RPA)
  is_b = wid >= NA
  rpw = jnp.where(is_b, RPB, RPA)
  iota = lax.iota(jnp.int32, L)
  zeros = jnp.zeros((L,), jnp.int32)

  tab = big_buf
  pltpu.sync_copy(idx, u_buf)

  def _zero(j, _):
    llist[pl.ds(j * L, L)] = zeros
    plist[pl.ds(j * L, L)] = zeros
    return 0
  lax.fori_loop(0, LISTN // L, _zero, 0)

  def _filter(k, cnt):
    v = u_buf[pl.ds(k * L, L)]
    inr = jnp.logical_and(v >= base, v < base + rpw)
    pos = k * L + iota
    inr_i = inr.astype(jnp.int32)
    cum = plsc.cumsum(inr_i)
    offs = cnt + cum - inr_i
    plsc.store_scatter(llist, [offs], v - base, mask=inr)
    plsc.store_scatter(plist, [offs], pos, mask=inr)
    return jnp.minimum(cnt + cum[L - 1], CAPL)
  n = lax.fori_loop(0, NIDX, _filter, jnp.int32(0))

  def _ded1(g, _):
    lanes = g * L + iota
    valid = lanes < n
    iv = llist[pl.ds(g * L, L)]
    pv = plist[pl.ds(g * L, L)]
    for l in range(L):
      plsc.store_scatter(tab, [iv], pv,
                         mask=jnp.logical_and(valid, iota == l))
    return 0
  lax.fori_loop(0, (n + L - 1) // L, _ded1, 0)

  def _ded2(g, m):
    lanes = g * L + iota
    valid = lanes < n
    iv = llist[pl.ds(g * L, L)]
    pv = plist[pl.ds(g * L, L)]
    w = plsc.load_gather(tab, [iv], mask=valid)
    win = jnp.logical_and(valid, w == pv)
    win_i = win.astype(jnp.int32)
    cum = plsc.cumsum(win_i)
    offs = m + cum - win_i
    plsc.store_scatter(wl, [offs], iv, mask=win)
    plsc.store_scatter(wp, [offs], pv, mask=win)
    return jnp.minimum(m + cum[L - 1], CAPW)
  m = lax.fori_loop(0, (n + L - 1) // L, _ded2, jnp.int32(0))

  @pl.when(m > 0)
  def _():
    def _pack(g, _):
      lv = wl[pl.ds(g * L, L)]
      pv = wp[pl.ds(g * L, L)]
      pk[pl.ds(g * L, L)] = jnp.bitwise_or(lv * PSHIFT, pv)
      return 0
    lax.fori_loop(0, (m + L - 1) // L, _pack, 0)

    pltpu.sync_copy(y, u_buf)

    def _ygat(g, _):
      lanes = g * L + iota
      msk = lanes < m
      pv = wp[pl.ds(g * L, L)]
      ywin[pl.ds(g * L, L)] = plsc.load_gather(u_buf, [pv], mask=msk)
      return 0
    lax.fori_loop(0, (m + L - 1) // L, _ygat, 0)

  pltpu.sync_copy(pk.at[pl.ds(0, CAPW)], pkw.at[wid])
  mstage[pl.ds(0, L)] = jnp.where(iota == 0, m, 0)
  pltpu.sync_copy(mstage, mcnt.at[wid])

  by_buf = big_buf

  @pl.when(jnp.logical_not(is_b))
  def _():
    pltpu.sync_copy(by.at[pl.ds(base, RPA)], by_buf.at[pl.ds(0, RPA)])

  @pl.when(is_b)
  def _():
    pltpu.sync_copy(by.at[pl.ds(base, RPB)], by_buf.at[pl.ds(0, RPB)])

  @pl.when(m > 0)
  def _():
    def _appy(g, _):
      lanes = g * L + iota
      msk = lanes < m
      iv = wl[pl.ds(g * L, L)]
      yvv = ywin[pl.ds(g * L, L)]
      plsc.store_scatter(by_buf, [iv], yvv, mask=msk)
      return 0
    lax.fori_loop(0, (m + L - 1) // L, _appy, 0)

  @pl.when(jnp.logical_not(is_b))
  def _():
    pltpu.sync_copy(by_buf.at[pl.ds(0, RPA)], oby.at[pl.ds(base, RPA)])

  @pl.when(is_b)
  def _():
    pltpu.sync_copy(by_buf.at[pl.ds(0, RPB)], oby.at[pl.ds(base, RPB)])


_mesh = plsc.VectorSubcoreMesh(core_axis_name="c", subcore_axis_name="s",
                               num_cores=NC, num_subcores=NS)

_sc_call = pl.kernel(
    _sc_body,
    out_type=(jax.ShapeDtypeStruct((CAP,), jnp.int32),
              jax.ShapeDtypeStruct((NW, CAPW), jnp.int32),
              jax.ShapeDtypeStruct((NW, L), jnp.int32)),
    mesh=_mesh,
    compiler_params=pltpu.CompilerParams(needs_layout_passes=False),
    scratch_types=[
        pltpu.VMEM((LISTN,), jnp.int32),
        pltpu.VMEM((LISTN,), jnp.int32),
        pltpu.VMEM((LISTN,), jnp.int32),
        pltpu.VMEM((LISTN,), jnp.int32),
        pltpu.VMEM((LISTN,), jnp.int32),
        pltpu.VMEM((LISTN,), jnp.int32),
        pltpu.VMEM((B,), jnp.int32),
        pltpu.VMEM((RPB,), jnp.int32),
        pltpu.VMEM((L,), jnp.int32),
        pltpu.SemaphoreType.DMA,
    ],
)


def _tc_body(bx_hbm, x_hbm, pkw_sm, mcnt_sm, out_hbm, sem_d):
  for w in range(NW):
    base_w = w * RPA + max(w - NA, 0) * (RPB - RPA)
    mw = mcnt_sm[w, 0]

    def _issue(k, _):
      e = pkw_sm[w, k]
      p = jax.lax.rem(e, PSHIFT)
      r = jax.lax.div(e, PSHIFT)
      pltpu.make_async_copy(x_hbm.at[pl.ds(p, 1)],
                            out_hbm.at[pl.ds(base_w + r, 1)], sem_d).start()
      return 0
    lax.fori_loop(0, mw, _issue, 0)

    def _drain(k, _):
      e = pkw_sm[w, k]
      p = jax.lax.rem(e, PSHIFT)
      r = jax.lax.div(e, PSHIFT)
      pltpu.make_async_copy(x_hbm.at[pl.ds(p, 1)],
                            out_hbm.at[pl.ds(base_w + r, 1)], sem_d).wait()
      return 0
    lax.fori_loop(0, mw, _drain, 0)


_tc_call = pl.pallas_call(
    _tc_body,
    out_shape=jax.ShapeDtypeStruct((CAP, FEAT), jnp.float32),
    in_specs=[
        pl.BlockSpec(memory_space=pltpu.HBM),
        pl.BlockSpec(memory_space=pltpu.HBM),
        pl.BlockSpec(memory_space=pltpu.SMEM),
        pl.BlockSpec(memory_space=pltpu.SMEM),
    ],
    out_specs=pl.BlockSpec(memory_space=pltpu.HBM),
    input_output_aliases={0: 0},
    scratch_shapes=[pltpu.SemaphoreType.DMA],
)


def kernel(bx, by, x, y, idx):
  oby, pkw, mcnt = _sc_call(by, y, idx)
  obx = _tc_call(bx, x, pkw, mcnt)
  return obx, oby
